# Initial kernel scaffold; baseline (speedup 1.0000x reference)
import jax
import jax.numpy as jnp
from jax.experimental import pallas as pl


def kernel(x, edge_index, k_hop_edge_index, neg_adj, W1, b1, Wm1, bm1, Wm2, bm2, Wa, ba):
    raise NotImplementedError("write your pallas kernel here")


# speedup vs baseline: 23.7352x; 23.7352x over previous
import dataclasses
import functools

import jax
import jax.numpy as jnp
from jax import lax
from jax.experimental import pallas as pl
from jax.experimental.pallas import tpu as pltpu
from jax.experimental.pallas import tpu_sc as plsc

N = 10000
D = 128
NP = 10240
DUMP = 10008
NC, NS, L = 2, 16, 16
NW = NC * NS

E = 320000
E_PAD = 327680
G_E = E_PAD // (NW * 128)

S = 960000
S_PAD = 983040
G_S = S_PAD // (NW * 128)

ROWS_PER_TILE = NP // NS


def _mesh():
    return plsc.VectorSubcoreMesh(core_axis_name="c", subcore_axis_name="s")


def _deg_body(dst2_hbm, deg_hbm, idx_v, ones_v, zb_v, shared):
    c = lax.axis_index("c")
    s = lax.axis_index("s")

    @pl.loop(0, 8)
    def _(i):
        ones_v[pl.ds(i * L, L)] = jnp.ones((L,), jnp.float32)

    @pl.loop(0, ROWS_PER_TILE // L)
    def _(i):
        zb_v[pl.ds(i * L, L)] = jnp.zeros((L,), jnp.float32)

    pltpu.sync_copy(zb_v, shared.at[pl.ds(s * ROWS_PER_TILE, ROWS_PER_TILE)])
    plsc.subcore_barrier()
    gbase = (c * NS + s) * G_E
    pltpu.sync_copy(dst2_hbm.at[pl.ds(gbase, G_E)], idx_v)

    @pl.loop(0, G_E)
    def _(j):
        pltpu.sync_copy(ones_v, shared.at[idx_v.at[j]], add=True)

    plsc.subcore_barrier()
    sl = pl.ds(s * ROWS_PER_TILE, ROWS_PER_TILE)
    pltpu.sync_copy(shared.at[sl], deg_hbm.at[c, sl])


def _deg(dst2):
    f = functools.partial(
        pl.kernel,
        out_type=jax.ShapeDtypeStruct((NC, NP), jnp.float32),
        mesh=_mesh(),
        scratch_types=[
            pltpu.VMEM((G_E, 128), jnp.int32),
            pltpu.VMEM((128,), jnp.float32),
            pltpu.VMEM((ROWS_PER_TILE,), jnp.float32),
            pltpu.VMEM_SHARED((NP,), jnp.float32),
        ],
    )(_deg_body)
    return f(dst2)


def _msg_body(g_hbm, src2_hbm, dst2_hbm, acc_hbm, isrc_v, idst_v, rows_v, shared):
    c = lax.axis_index("c")
    s = lax.axis_index("s")

    @pl.loop(0, 128)
    def _(r):
        @pl.loop(0, D // L)
        def _(k):
            rows_v[r, pl.ds(k * L, L)] = jnp.zeros((L,), jnp.float32)

    @pl.loop(0, ROWS_PER_TILE // 128)
    def _(i):
        pltpu.sync_copy(rows_v, shared.at[pl.ds(s * ROWS_PER_TILE + i * 128, 128)])

    plsc.subcore_barrier()
    gbase = (c * NS + s) * G_E
    pltpu.sync_copy(src2_hbm.at[pl.ds(gbase, G_E)], isrc_v)
    pltpu.sync_copy(dst2_hbm.at[pl.ds(gbase, G_E)], idst_v)

    @pl.loop(0, G_E)
    def _(j):
        pltpu.sync_copy(g_hbm.at[isrc_v.at[j]], rows_v)
        pltpu.sync_copy(rows_v, shared.at[idst_v.at[j]], add=True)

    plsc.subcore_barrier()

    @pl.loop(0, ROWS_PER_TILE // 128)
    def _(i):
        sl = pl.ds(s * ROWS_PER_TILE + i * 128, 128)
        pltpu.sync_copy(shared.at[sl], acc_hbm.at[c, sl])


def _msg(g, src2, dst2):
    f = functools.partial(
        pl.kernel,
        out_type=jax.ShapeDtypeStruct((NC, NP, D), jnp.float32),
        mesh=_mesh(),
        scratch_types=[
            pltpu.VMEM((G_E, 128), jnp.int32),
            pltpu.VMEM((G_E, 128), jnp.int32),
            pltpu.VMEM((128, D), jnp.float32),
            pltpu.VMEM_SHARED((NP, D), jnp.float32),
        ],
    )(_msg_body)
    return f(g, src2, dst2)


def _score_body(p_hbm, q_hbm, a2_hbm, b2_hbm, adj_hbm, p_v, q_v, ia_v, ib_v, out_v):
    c = lax.axis_index("c")
    s = lax.axis_index("s")
    pltpu.sync_copy(p_hbm, p_v)
    pltpu.sync_copy(q_hbm, q_v)
    gbase = (c * NS + s) * G_S
    pltpu.sync_copy(a2_hbm.at[pl.ds(gbase, G_S)], ia_v)
    pltpu.sync_copy(b2_hbm.at[pl.ds(gbase, G_S)], ib_v)

    @pl.loop(0, G_S)
    def _(j):
        for k in range(128 // L):
            sl = pl.ds(k * L, L)
            va = plsc.load_gather(p_v, [ia_v[j, sl]])
            vb = plsc.load_gather(q_v, [ib_v[j, sl]])
            out_v[j, sl] = 1.0 / (1.0 + jnp.exp(-(va + vb)))

    pltpu.sync_copy(out_v, adj_hbm.at[pl.ds(gbase, G_S)])


def _score(p, q, a2, b2):
    cp = pltpu.CompilerParams()
    if "needs_layout_passes" in pltpu.CompilerParams.__dataclass_fields__:
        cp = dataclasses.replace(cp, needs_layout_passes=False)
    f = functools.partial(
        pl.kernel,
        out_type=jax.ShapeDtypeStruct((S_PAD // 128, 128), jnp.float32),
        mesh=_mesh(),
        compiler_params=cp,
        scratch_types=[
            pltpu.VMEM((N,), jnp.float32),
            pltpu.VMEM((N,), jnp.float32),
            pltpu.VMEM((G_S, 128), jnp.int32),
            pltpu.VMEM((G_S, 128), jnp.int32),
            pltpu.VMEM((G_S, 128), jnp.float32),
        ],
    )(_score_body)
    return f(p, q, a2, b2)


_BLK = 2000


def _mm_body(x_ref, w_ref, o_ref):
    o_ref[...] = jnp.dot(x_ref[...], w_ref[...], preferred_element_type=jnp.float32)


def _mm(x, w):
    return pl.pallas_call(
        _mm_body,
        grid=(N // _BLK,),
        in_specs=[
            pl.BlockSpec((_BLK, D), lambda i: (i, 0)),
            pl.BlockSpec((D, D), lambda i: (0, 0)),
        ],
        out_specs=pl.BlockSpec((_BLK, D), lambda i: (i, 0)),
        out_shape=jax.ShapeDtypeStruct((N, D), jnp.float32),
    )(x, w)


def _gscale_body(h_ref, d0_ref, d1_ref, g_ref, dinv_ref):
    deg = d0_ref[...] + d1_ref[...] + 1.0
    dinv = lax.rsqrt(deg)
    dinv_ref[...] = dinv
    g_ref[...] = h_ref[...] * dinv


def _gscale(h, d0, d1):
    return pl.pallas_call(
        _gscale_body,
        grid=(N // _BLK,),
        in_specs=[
            pl.BlockSpec((_BLK, D), lambda i: (i, 0)),
            pl.BlockSpec((_BLK, 1), lambda i: (i, 0)),
            pl.BlockSpec((_BLK, 1), lambda i: (i, 0)),
        ],
        out_specs=[
            pl.BlockSpec((_BLK, D), lambda i: (i, 0)),
            pl.BlockSpec((_BLK, 1), lambda i: (i, 0)),
        ],
        out_shape=[
            jax.ShapeDtypeStruct((N, D), jnp.float32),
            jax.ShapeDtypeStruct((N, 1), jnp.float32),
        ],
    )(h, d0, d1)


def _head_body(a0_ref, a1_ref, h_ref, dinv_ref, b1_ref, wm1_ref, bm1_ref,
               wm2_ref, bm2_ref, wa1_ref, wa2_ref, ba_ref,
               f_ref, p_ref, q_ref):
    dinv = dinv_ref[...]
    out = dinv * (a0_ref[...] + a1_ref[...]) + (dinv * dinv) * h_ref[...] + b1_ref[...]
    t = jnp.maximum(
        jnp.dot(out, wm1_ref[...], preferred_element_type=jnp.float32) + bm1_ref[...],
        0.0,
    )
    f_ref[...] = jax.nn.sigmoid(
        jnp.dot(t, wm2_ref[...], preferred_element_type=jnp.float32) + bm2_ref[...]
    )
    p_ref[...] = jnp.dot(out, wa1_ref[...], preferred_element_type=jnp.float32) + ba_ref[0, 0]
    q_ref[...] = jnp.dot(out, wa2_ref[...], preferred_element_type=jnp.float32)


def _head(a0, a1, h, dinv, b1, wm1, bm1, wm2, bm2, wa1, wa2, ba):
    full = lambda shape: pl.BlockSpec(shape, lambda i: tuple(0 for _ in shape))
    return pl.pallas_call(
        _head_body,
        grid=(N // _BLK,),
        in_specs=[
            pl.BlockSpec((_BLK, D), lambda i: (i, 0)),
            pl.BlockSpec((_BLK, D), lambda i: (i, 0)),
            pl.BlockSpec((_BLK, D), lambda i: (i, 0)),
            pl.BlockSpec((_BLK, 1), lambda i: (i, 0)),
            full((1, D)),
            full((D, D)),
            full((1, D)),
            full((D, D)),
            full((1, D)),
            full((D, 1)),
            full((D, 1)),
            full((1, 1)),
        ],
        out_specs=[
            pl.BlockSpec((_BLK, D), lambda i: (i, 0)),
            pl.BlockSpec((_BLK, 1), lambda i: (i, 0)),
            pl.BlockSpec((_BLK, 1), lambda i: (i, 0)),
        ],
        out_shape=[
            jax.ShapeDtypeStruct((N, D), jnp.float32),
            jax.ShapeDtypeStruct((N, 1), jnp.float32),
            jax.ShapeDtypeStruct((N, 1), jnp.float32),
        ],
    )(a0, a1, h, dinv, b1, wm1, bm1, wm2, bm2, wa1, wa2, ba)


def kernel(x, edge_index, k_hop_edge_index, neg_adj, W1, b1, Wm1, bm1, Wm2, bm2, Wa, ba):
    src = edge_index[0]
    dst = edge_index[1]
    epad = E_PAD - E
    src2 = jnp.concatenate([src, jnp.zeros((epad,), jnp.int32)]).reshape(-1, 128)
    dst2 = jnp.concatenate([dst, jnp.full((epad,), DUMP, jnp.int32)]).reshape(-1, 128)
    spad = S_PAD - S
    a2 = jnp.concatenate(
        [k_hop_edge_index[0], neg_adj[:, 0], jnp.zeros((spad,), jnp.int32)]
    ).reshape(-1, 128)
    b2 = jnp.concatenate(
        [k_hop_edge_index[1], neg_adj[:, 1], jnp.zeros((spad,), jnp.int32)]
    ).reshape(-1, 128)

    h = _mm(x, W1)
    deg_parts = _deg(dst2)
    d0 = deg_parts[0].reshape(NP, 1)
    d1 = deg_parts[1].reshape(NP, 1)
    g, dinv = _gscale(h, d0[:N], d1[:N])
    acc = _msg(g, src2, dst2)
    f, p, q = _head(
        acc[0], acc[1], h, dinv,
        b1.reshape(1, D), Wm1, bm1.reshape(1, D), Wm2, bm2.reshape(1, D),
        Wa[:D], Wa[D:], ba.reshape(1, 1),
    )
    adj2 = _score(p.reshape(-1), q.reshape(-1), a2, b2)
    adj_out = adj2.reshape(-1)[:S]
    return f, adj_out


# speedup vs baseline: 25.4035x; 1.0703x over previous
import dataclasses
import functools

import jax
import jax.numpy as jnp
from jax import lax
from jax.experimental import pallas as pl
from jax.experimental.pallas import tpu as pltpu
from jax.experimental.pallas import tpu_sc as plsc

N = 10000
D = 128
NP = 10240
DUMP = 10008
NC, NS, L = 2, 16, 16
NW = NC * NS

E = 320000
E_PAD = 327680
G_E = E_PAD // (NW * 128)

S = 960000
S_PAD = 983040
G_S = S_PAD // (NW * 128)

ROWS_PER_TILE = NP // NS


def _mesh():
    return plsc.VectorSubcoreMesh(core_axis_name="c", subcore_axis_name="s")


def _deg_body(dst2_hbm, deg_hbm, idx_v, ones_v, zb_v, shared):
    c = lax.axis_index("c")
    s = lax.axis_index("s")

    @pl.loop(0, 8)
    def _(i):
        ones_v[pl.ds(i * L, L)] = jnp.ones((L,), jnp.float32)

    @pl.loop(0, ROWS_PER_TILE // L)
    def _(i):
        zb_v[pl.ds(i * L, L)] = jnp.zeros((L,), jnp.float32)

    pltpu.sync_copy(zb_v, shared.at[pl.ds(s * ROWS_PER_TILE, ROWS_PER_TILE)])
    plsc.subcore_barrier()
    gbase = (c * NS + s) * G_E
    pltpu.sync_copy(dst2_hbm.at[pl.ds(gbase, G_E)], idx_v)

    @pl.loop(0, G_E)
    def _(j):
        pltpu.sync_copy(ones_v, shared.at[idx_v.at[j]], add=True)

    plsc.subcore_barrier()
    sl = pl.ds(s * ROWS_PER_TILE, ROWS_PER_TILE)
    pltpu.sync_copy(shared.at[sl], deg_hbm.at[c, sl])


def _deg(dst2):
    f = functools.partial(
        pl.kernel,
        out_type=jax.ShapeDtypeStruct((NC, NP), jnp.float32),
        mesh=_mesh(),
        scratch_types=[
            pltpu.VMEM((G_E, 128), jnp.int32),
            pltpu.VMEM((128,), jnp.float32),
            pltpu.VMEM((ROWS_PER_TILE,), jnp.float32),
            pltpu.VMEM_SHARED((NP,), jnp.float32),
        ],
    )(_deg_body)
    return f(dst2)


_HALF = G_E // 2


def _msg_body(g_hbm, src2_hbm, dst2_hbm, acc_hbm, isrc_v, idst_v, rows_a, rows_b,
              shared, sem_ga, sem_gb, sem_sa, sem_sb):
    c = lax.axis_index("c")
    s = lax.axis_index("s")

    @pl.loop(0, 128)
    def _(r):
        @pl.loop(0, D // L)
        def _(k):
            rows_a[r, pl.ds(k * L, L)] = jnp.zeros((L,), jnp.float32)

    @pl.loop(0, ROWS_PER_TILE // 128)
    def _(i):
        pltpu.sync_copy(rows_a, shared.at[pl.ds(s * ROWS_PER_TILE + i * 128, 128)])

    plsc.subcore_barrier()
    gbase = (c * NS + s) * G_E

    def wait_gather(buf, sem):
        pltpu.make_async_copy(g_hbm.at[isrc_v.at[0]], buf, sem).wait()

    def wait_scatter(buf, sem):
        pltpu.make_async_copy(buf, shared.at[idst_v.at[0]], sem).wait()

    for half in range(2):
        base = gbase + half * _HALF
        pltpu.sync_copy(src2_hbm.at[pl.ds(base, _HALF)], isrc_v)
        pltpu.sync_copy(dst2_hbm.at[pl.ds(base, _HALF)], idst_v)
        pltpu.async_copy(g_hbm.at[isrc_v.at[0]], rows_a, sem_ga)

        @pl.loop(0, _HALF // 2)
        def _(i):
            j = 2 * i
            wait_gather(rows_a, sem_ga)

            @pl.when(i >= 1)
            def _():
                wait_scatter(rows_b, sem_sb)

            pltpu.async_copy(g_hbm.at[isrc_v.at[j + 1]], rows_b, sem_gb)
            pltpu.async_copy(rows_a, shared.at[idst_v.at[j]], sem_sa, add=True)
            wait_gather(rows_b, sem_gb)
            wait_scatter(rows_a, sem_sa)

            @pl.when(j + 2 < _HALF)
            def _():
                pltpu.async_copy(g_hbm.at[isrc_v.at[j + 2]], rows_a, sem_ga)

            pltpu.async_copy(rows_b, shared.at[idst_v.at[j + 1]], sem_sb, add=True)

        wait_scatter(rows_b, sem_sb)

    plsc.subcore_barrier()

    @pl.loop(0, ROWS_PER_TILE // 128)
    def _(i):
        sl = pl.ds(s * ROWS_PER_TILE + i * 128, 128)
        pltpu.sync_copy(shared.at[sl], acc_hbm.at[c, sl])


def _msg(g, src2, dst2):
    f = functools.partial(
        pl.kernel,
        out_type=jax.ShapeDtypeStruct((NC, NP, D), jnp.float32),
        mesh=_mesh(),
        scratch_types=[
            pltpu.VMEM((_HALF, 128), jnp.int32),
            pltpu.VMEM((_HALF, 128), jnp.int32),
            pltpu.VMEM((128, D), jnp.float32),
            pltpu.VMEM((128, D), jnp.float32),
            pltpu.VMEM_SHARED((NP, D), jnp.float32),
            pltpu.SemaphoreType.DMA,
            pltpu.SemaphoreType.DMA,
            pltpu.SemaphoreType.DMA,
            pltpu.SemaphoreType.DMA,
        ],
    )(_msg_body)
    return f(g, src2, dst2)


def _score_body(p_hbm, q_hbm, a2_hbm, b2_hbm, adj_hbm, p_v, q_v, ia_v, ib_v, out_v):
    c = lax.axis_index("c")
    s = lax.axis_index("s")
    pltpu.sync_copy(p_hbm, p_v)
    pltpu.sync_copy(q_hbm, q_v)
    gbase = (c * NS + s) * G_S
    pltpu.sync_copy(a2_hbm.at[pl.ds(gbase, G_S)], ia_v)
    pltpu.sync_copy(b2_hbm.at[pl.ds(gbase, G_S)], ib_v)

    @pl.loop(0, G_S)
    def _(j):
        for k in range(128 // L):
            sl = pl.ds(k * L, L)
            va = plsc.load_gather(p_v, [ia_v[j, sl]])
            vb = plsc.load_gather(q_v, [ib_v[j, sl]])
            out_v[j, sl] = 1.0 / (1.0 + jnp.exp(-(va + vb)))

    pltpu.sync_copy(out_v, adj_hbm.at[pl.ds(gbase, G_S)])


def _score(p, q, a2, b2):
    cp = pltpu.CompilerParams()
    if "needs_layout_passes" in pltpu.CompilerParams.__dataclass_fields__:
        cp = dataclasses.replace(cp, needs_layout_passes=False)
    f = functools.partial(
        pl.kernel,
        out_type=jax.ShapeDtypeStruct((S_PAD // 128, 128), jnp.float32),
        mesh=_mesh(),
        compiler_params=cp,
        scratch_types=[
            pltpu.VMEM((N,), jnp.float32),
            pltpu.VMEM((N,), jnp.float32),
            pltpu.VMEM((G_S, 128), jnp.int32),
            pltpu.VMEM((G_S, 128), jnp.int32),
            pltpu.VMEM((G_S, 128), jnp.float32),
        ],
    )(_score_body)
    return f(p, q, a2, b2)


_BLK = 2000


def _mm_body(x_ref, w_ref, o_ref):
    o_ref[...] = jnp.dot(x_ref[...], w_ref[...], preferred_element_type=jnp.float32)


def _mm(x, w):
    return pl.pallas_call(
        _mm_body,
        grid=(N // _BLK,),
        in_specs=[
            pl.BlockSpec((_BLK, D), lambda i: (i, 0)),
            pl.BlockSpec((D, D), lambda i: (0, 0)),
        ],
        out_specs=pl.BlockSpec((_BLK, D), lambda i: (i, 0)),
        out_shape=jax.ShapeDtypeStruct((N, D), jnp.float32),
    )(x, w)


def _gscale_body(h_ref, d0_ref, d1_ref, g_ref, dinv_ref):
    deg = d0_ref[...] + d1_ref[...] + 1.0
    dinv = lax.rsqrt(deg)
    dinv_ref[...] = dinv
    g_ref[...] = h_ref[...] * dinv


def _gscale(h, d0, d1):
    return pl.pallas_call(
        _gscale_body,
        grid=(N // _BLK,),
        in_specs=[
            pl.BlockSpec((_BLK, D), lambda i: (i, 0)),
            pl.BlockSpec((_BLK, 1), lambda i: (i, 0)),
            pl.BlockSpec((_BLK, 1), lambda i: (i, 0)),
        ],
        out_specs=[
            pl.BlockSpec((_BLK, D), lambda i: (i, 0)),
            pl.BlockSpec((_BLK, 1), lambda i: (i, 0)),
        ],
        out_shape=[
            jax.ShapeDtypeStruct((N, D), jnp.float32),
            jax.ShapeDtypeStruct((N, 1), jnp.float32),
        ],
    )(h, d0, d1)


def _head_body(a0_ref, a1_ref, h_ref, dinv_ref, b1_ref, wm1_ref, bm1_ref,
               wm2_ref, bm2_ref, wa1_ref, wa2_ref, ba_ref,
               f_ref, p_ref, q_ref):
    dinv = dinv_ref[...]
    out = dinv * (a0_ref[...] + a1_ref[...]) + (dinv * dinv) * h_ref[...] + b1_ref[...]
    t = jnp.maximum(
        jnp.dot(out, wm1_ref[...], preferred_element_type=jnp.float32) + bm1_ref[...],
        0.0,
    )
    f_ref[...] = jax.nn.sigmoid(
        jnp.dot(t, wm2_ref[...], preferred_element_type=jnp.float32) + bm2_ref[...]
    )
    p_ref[...] = jnp.dot(out, wa1_ref[...], preferred_element_type=jnp.float32) + ba_ref[0, 0]
    q_ref[...] = jnp.dot(out, wa2_ref[...], preferred_element_type=jnp.float32)


def _head(a0, a1, h, dinv, b1, wm1, bm1, wm2, bm2, wa1, wa2, ba):
    full = lambda shape: pl.BlockSpec(shape, lambda i: tuple(0 for _ in shape))
    return pl.pallas_call(
        _head_body,
        grid=(N // _BLK,),
        in_specs=[
            pl.BlockSpec((_BLK, D), lambda i: (i, 0)),
            pl.BlockSpec((_BLK, D), lambda i: (i, 0)),
            pl.BlockSpec((_BLK, D), lambda i: (i, 0)),
            pl.BlockSpec((_BLK, 1), lambda i: (i, 0)),
            full((1, D)),
            full((D, D)),
            full((1, D)),
            full((D, D)),
            full((1, D)),
            full((D, 1)),
            full((D, 1)),
            full((1, 1)),
        ],
        out_specs=[
            pl.BlockSpec((_BLK, D), lambda i: (i, 0)),
            pl.BlockSpec((_BLK, 1), lambda i: (i, 0)),
            pl.BlockSpec((_BLK, 1), lambda i: (i, 0)),
        ],
        out_shape=[
            jax.ShapeDtypeStruct((N, D), jnp.float32),
            jax.ShapeDtypeStruct((N, 1), jnp.float32),
            jax.ShapeDtypeStruct((N, 1), jnp.float32),
        ],
    )(a0, a1, h, dinv, b1, wm1, bm1, wm2, bm2, wa1, wa2, ba)


def kernel(x, edge_index, k_hop_edge_index, neg_adj, W1, b1, Wm1, bm1, Wm2, bm2, Wa, ba):
    src = edge_index[0]
    dst = edge_index[1]
    epad = E_PAD - E
    src2 = jnp.concatenate([src, jnp.zeros((epad,), jnp.int32)]).reshape(-1, 128)
    dst2 = jnp.concatenate([dst, jnp.full((epad,), DUMP, jnp.int32)]).reshape(-1, 128)
    spad = S_PAD - S
    a2 = jnp.concatenate(
        [k_hop_edge_index[0], neg_adj[:, 0], jnp.zeros((spad,), jnp.int32)]
    ).reshape(-1, 128)
    b2 = jnp.concatenate(
        [k_hop_edge_index[1], neg_adj[:, 1], jnp.zeros((spad,), jnp.int32)]
    ).reshape(-1, 128)

    h = _mm(x, W1)
    deg_parts = _deg(dst2)
    d0 = deg_parts[0].reshape(NP, 1)
    d1 = deg_parts[1].reshape(NP, 1)
    g, dinv = _gscale(h, d0[:N], d1[:N])
    acc = _msg(g, src2, dst2)
    f, p, q = _head(
        acc[0], acc[1], h, dinv,
        b1.reshape(1, D), Wm1, bm1.reshape(1, D), Wm2, bm2.reshape(1, D),
        Wa[:D], Wa[D:], ba.reshape(1, 1),
    )
    adj2 = _score(p.reshape(-1), q.reshape(-1), a2, b2)
    adj_out = adj2.reshape(-1)[:S]
    return f, adj_out


# speedup vs baseline: 51.6767x; 2.0342x over previous
import dataclasses
import functools

import jax
import jax.numpy as jnp
from jax import lax
from jax.experimental import pallas as pl
from jax.experimental.pallas import tpu as pltpu
from jax.experimental.pallas import tpu_sc as plsc

N = 10000
D = 128
NP = 10240
DUMP = 10008
NC, NS, L = 2, 16, 16
NW = NC * NS

E = 320000
E_PAD = 327680
G_E = E_PAD // (NW * 128)

S = 960000
S_PAD = 983040
G_S = S_PAD // (NW * 128)

ROWS_PER_TILE = NP // NS


def _mesh():
    return plsc.VectorSubcoreMesh(core_axis_name="c", subcore_axis_name="s")


def _deg_body(dst2_hbm, deg_hbm, idx_v, ones_v, zb_v, shared):
    c = lax.axis_index("c")
    s = lax.axis_index("s")

    @pl.loop(0, 8)
    def _(i):
        ones_v[pl.ds(i * L, L)] = jnp.ones((L,), jnp.float32)

    @pl.loop(0, ROWS_PER_TILE // L)
    def _(i):
        zb_v[pl.ds(i * L, L)] = jnp.zeros((L,), jnp.float32)

    pltpu.sync_copy(zb_v, shared.at[pl.ds(s * ROWS_PER_TILE, ROWS_PER_TILE)])
    plsc.subcore_barrier()
    gbase = (c * NS + s) * G_E
    pltpu.sync_copy(dst2_hbm.at[pl.ds(gbase, G_E)], idx_v)

    @pl.loop(0, G_E)
    def _(j):
        pltpu.sync_copy(ones_v, shared.at[idx_v.at[j]], add=True)

    plsc.subcore_barrier()
    sl = pl.ds(s * ROWS_PER_TILE, ROWS_PER_TILE)
    pltpu.sync_copy(shared.at[sl], deg_hbm.at[c, sl])


def _deg(dst2):
    f = functools.partial(
        pl.kernel,
        out_type=jax.ShapeDtypeStruct((NC, NP), jnp.float32),
        mesh=_mesh(),
        scratch_types=[
            pltpu.VMEM((G_E, 128), jnp.int32),
            pltpu.VMEM((128,), jnp.float32),
            pltpu.VMEM((ROWS_PER_TILE,), jnp.float32),
            pltpu.VMEM_SHARED((NP,), jnp.float32),
        ],
    )(_deg_body)
    return f(dst2)


_HALF = G_E // 2


def _msg_body(g_hbm, src2_hbm, dst2_hbm, acc_hbm, isrc_v, idst_v, rows_a, rows_b,
              shared, sem_ga, sem_gb, sem_sa, sem_sb):
    c = lax.axis_index("c")
    s = lax.axis_index("s")

    @pl.loop(0, 128)
    def _(r):
        @pl.loop(0, D // L)
        def _(k):
            rows_a[r, pl.ds(k * L, L)] = jnp.zeros((L,), jnp.float32)

    @pl.loop(0, ROWS_PER_TILE // 128)
    def _(i):
        pltpu.sync_copy(rows_a, shared.at[pl.ds(s * ROWS_PER_TILE + i * 128, 128)])

    plsc.subcore_barrier()
    gbase = (c * NS + s) * G_E

    def wait_gather(buf, sem):
        pltpu.make_async_copy(g_hbm.at[isrc_v.at[0]], buf, sem).wait()

    def wait_scatter(buf, sem):
        pltpu.make_async_copy(buf, shared.at[idst_v.at[0]], sem).wait()

    for half in range(2):
        base = gbase + half * _HALF
        pltpu.sync_copy(src2_hbm.at[pl.ds(base, _HALF)], isrc_v)
        pltpu.sync_copy(dst2_hbm.at[pl.ds(base, _HALF)], idst_v)
        pltpu.async_copy(g_hbm.at[isrc_v.at[0]], rows_a, sem_ga)

        @pl.loop(0, _HALF // 2)
        def _(i):
            j = 2 * i
            wait_gather(rows_a, sem_ga)

            @pl.when(i >= 1)
            def _():
                wait_scatter(rows_b, sem_sb)

            pltpu.async_copy(g_hbm.at[isrc_v.at[j + 1]], rows_b, sem_gb)
            pltpu.async_copy(rows_a, shared.at[idst_v.at[j]], sem_sa, add=True)
            wait_gather(rows_b, sem_gb)
            wait_scatter(rows_a, sem_sa)

            @pl.when(j + 2 < _HALF)
            def _():
                pltpu.async_copy(g_hbm.at[isrc_v.at[j + 2]], rows_a, sem_ga)

            pltpu.async_copy(rows_b, shared.at[idst_v.at[j + 1]], sem_sb, add=True)

        wait_scatter(rows_b, sem_sb)

    plsc.subcore_barrier()

    @pl.loop(0, ROWS_PER_TILE // 128)
    def _(i):
        sl = pl.ds(s * ROWS_PER_TILE + i * 128, 128)
        pltpu.sync_copy(shared.at[sl], acc_hbm.at[c, sl])


def _msg(g, src2, dst2):
    f = functools.partial(
        pl.kernel,
        out_type=jax.ShapeDtypeStruct((NC, NP, D), jnp.float32),
        mesh=_mesh(),
        scratch_types=[
            pltpu.VMEM((_HALF, 128), jnp.int32),
            pltpu.VMEM((_HALF, 128), jnp.int32),
            pltpu.VMEM((128, D), jnp.float32),
            pltpu.VMEM((128, D), jnp.float32),
            pltpu.VMEM_SHARED((NP, D), jnp.float32),
            pltpu.SemaphoreType.DMA,
            pltpu.SemaphoreType.DMA,
            pltpu.SemaphoreType.DMA,
            pltpu.SemaphoreType.DMA,
        ],
    )(_msg_body)
    return f(g, src2, dst2)


def _score_body(p_hbm, q_hbm, a2_hbm, b2_hbm, adj_hbm, p_v, q_v, ia_v, ib_v, out_v):
    c = lax.axis_index("c")
    s = lax.axis_index("s")
    pltpu.sync_copy(p_hbm, p_v)
    pltpu.sync_copy(q_hbm, q_v)
    gbase = (c * NS + s) * G_S
    pltpu.sync_copy(a2_hbm.at[pl.ds(gbase, G_S)], ia_v)
    pltpu.sync_copy(b2_hbm.at[pl.ds(gbase, G_S)], ib_v)

    @pl.loop(0, G_S)
    def _(j):
        for k in range(128 // L):
            sl = pl.ds(k * L, L)
            va = plsc.load_gather(p_v, [ia_v[j, sl]])
            vb = plsc.load_gather(q_v, [ib_v[j, sl]])
            out_v[j, sl] = 1.0 / (1.0 + jnp.exp(-(va + vb)))

    pltpu.sync_copy(out_v, adj_hbm.at[pl.ds(gbase, G_S)])


def _score(p, q, a2, b2):
    cp = pltpu.CompilerParams()
    if "needs_layout_passes" in pltpu.CompilerParams.__dataclass_fields__:
        cp = dataclasses.replace(cp, needs_layout_passes=False)
    f = functools.partial(
        pl.kernel,
        out_type=jax.ShapeDtypeStruct((S_PAD // 128, 128), jnp.float32),
        mesh=_mesh(),
        compiler_params=cp,
        scratch_types=[
            pltpu.VMEM((N,), jnp.float32),
            pltpu.VMEM((N,), jnp.float32),
            pltpu.VMEM((G_S, 128), jnp.int32),
            pltpu.VMEM((G_S, 128), jnp.int32),
            pltpu.VMEM((G_S, 128), jnp.float32),
        ],
    )(_score_body)
    return f(p, q, a2, b2)


_BLK = 2000


def _mm_body(x_ref, w_ref, o_ref):
    o_ref[...] = jnp.dot(x_ref[...], w_ref[...], preferred_element_type=jnp.float32)


def _mm(x, w):
    return pl.pallas_call(
        _mm_body,
        grid=(N // _BLK,),
        in_specs=[
            pl.BlockSpec((_BLK, D), lambda i: (i, 0)),
            pl.BlockSpec((D, D), lambda i: (0, 0)),
        ],
        out_specs=pl.BlockSpec((_BLK, D), lambda i: (i, 0)),
        out_shape=jax.ShapeDtypeStruct((N, D), jnp.float32),
    )(x, w)


def _gscale_body(h_ref, d0_ref, d1_ref, g_ref, dinv_ref):
    deg = d0_ref[...] + d1_ref[...] + 1.0
    dinv = lax.rsqrt(deg)
    dinv_ref[...] = dinv
    g_ref[...] = h_ref[...] * dinv


def _gscale(h, d0, d1):
    return pl.pallas_call(
        _gscale_body,
        grid=(N // _BLK,),
        in_specs=[
            pl.BlockSpec((_BLK, D), lambda i: (i, 0)),
            pl.BlockSpec((_BLK, 1), lambda i: (i, 0)),
            pl.BlockSpec((_BLK, 1), lambda i: (i, 0)),
        ],
        out_specs=[
            pl.BlockSpec((_BLK, D), lambda i: (i, 0)),
            pl.BlockSpec((_BLK, 1), lambda i: (i, 0)),
        ],
        out_shape=[
            jax.ShapeDtypeStruct((N, D), jnp.float32),
            jax.ShapeDtypeStruct((N, 1), jnp.float32),
        ],
    )(h, d0, d1)


def _head_body(a0_ref, a1_ref, h_ref, dinv_ref, b1_ref, wm1_ref, bm1_ref,
               wm2_ref, bm2_ref, wa1_ref, wa2_ref, ba_ref,
               f_ref, p_ref, q_ref):
    dinv = dinv_ref[...]
    out = dinv * (a0_ref[...] + a1_ref[...]) + (dinv * dinv) * h_ref[...] + b1_ref[...]
    t = jnp.maximum(
        jnp.dot(out, wm1_ref[...], preferred_element_type=jnp.float32) + bm1_ref[...],
        0.0,
    )
    f_ref[...] = jax.nn.sigmoid(
        jnp.dot(t, wm2_ref[...], preferred_element_type=jnp.float32) + bm2_ref[...]
    )
    p_ref[...] = jnp.dot(out, wa1_ref[...], preferred_element_type=jnp.float32) + ba_ref[0, 0]
    q_ref[...] = jnp.dot(out, wa2_ref[...], preferred_element_type=jnp.float32)


def _head(a0, a1, h, dinv, b1, wm1, bm1, wm2, bm2, wa1, wa2, ba):
    full = lambda shape: pl.BlockSpec(shape, lambda i: tuple(0 for _ in shape))
    return pl.pallas_call(
        _head_body,
        grid=(N // _BLK,),
        in_specs=[
            pl.BlockSpec((_BLK, D), lambda i: (i, 0)),
            pl.BlockSpec((_BLK, D), lambda i: (i, 0)),
            pl.BlockSpec((_BLK, D), lambda i: (i, 0)),
            pl.BlockSpec((_BLK, 1), lambda i: (i, 0)),
            full((1, D)),
            full((D, D)),
            full((1, D)),
            full((D, D)),
            full((1, D)),
            full((D, 1)),
            full((D, 1)),
            full((1, 1)),
        ],
        out_specs=[
            pl.BlockSpec((_BLK, D), lambda i: (i, 0)),
            pl.BlockSpec((_BLK, 1), lambda i: (i, 0)),
            pl.BlockSpec((_BLK, 1), lambda i: (i, 0)),
        ],
        out_shape=[
            jax.ShapeDtypeStruct((N, D), jnp.float32),
            jax.ShapeDtypeStruct((N, 1), jnp.float32),
            jax.ShapeDtypeStruct((N, 1), jnp.float32),
        ],
    )(a0, a1, h, dinv, b1, wm1, bm1, wm2, bm2, wa1, wa2, ba)


def kernel(x, edge_index, k_hop_edge_index, neg_adj, W1, b1, Wm1, bm1, Wm2, bm2, Wa, ba):
    src = edge_index[0]
    dst = edge_index[1]
    epad = E_PAD - E
    pad_src = jnp.arange(epad, dtype=jnp.int32) % N
    pad_dst = N + jnp.arange(epad, dtype=jnp.int32) % (NP - N)
    src2 = jnp.concatenate([src, pad_src]).reshape(-1, 128)
    dst2 = jnp.concatenate([dst, pad_dst]).reshape(-1, 128)
    spad = S_PAD - S
    pad_ab = jnp.arange(spad, dtype=jnp.int32) % N
    a2 = jnp.concatenate(
        [k_hop_edge_index[0], neg_adj[:, 0], pad_ab]
    ).reshape(-1, 128)
    b2 = jnp.concatenate(
        [k_hop_edge_index[1], neg_adj[:, 1], pad_ab]
    ).reshape(-1, 128)

    h = _mm(x, W1)
    deg_parts = _deg(dst2)
    d0 = deg_parts[0].reshape(NP, 1)
    d1 = deg_parts[1].reshape(NP, 1)
    g, dinv = _gscale(h, d0[:N], d1[:N])
    acc = _msg(g, src2, dst2)
    f, p, q = _head(
        acc[0], acc[1], h, dinv,
        b1.reshape(1, D), Wm1, bm1.reshape(1, D), Wm2, bm2.reshape(1, D),
        Wa[:D], Wa[D:], ba.reshape(1, 1),
    )
    adj2 = _score(p.reshape(-1), q.reshape(-1), a2, b2)
    adj_out = adj2.reshape(-1)[:S]
    return f, adj_out


# speedup vs baseline: 51.6871x; 1.0002x over previous
import dataclasses
import functools

import jax
import jax.numpy as jnp
from jax import lax
from jax.experimental import pallas as pl
from jax.experimental.pallas import tpu as pltpu
from jax.experimental.pallas import tpu_sc as plsc

N = 10000
D = 128
NP = 10240
DUMP = 10008
NC, NS, L = 2, 16, 16
NW = NC * NS

E = 320000
E_PAD = 327680
G_E = E_PAD // (NW * 128)

S = 960000
S_PAD = 983040
G_S = S_PAD // (NW * 128)

ROWS_PER_TILE = NP // NS


def _mesh():
    return plsc.VectorSubcoreMesh(core_axis_name="c", subcore_axis_name="s")


def _deg_body(dst2_hbm, deg_hbm, idx_v, ones_v, zb_v, shared):
    c = lax.axis_index("c")
    s = lax.axis_index("s")

    @pl.loop(0, 8)
    def _(i):
        ones_v[pl.ds(i * L, L)] = jnp.ones((L,), jnp.float32)

    @pl.loop(0, ROWS_PER_TILE // L)
    def _(i):
        zb_v[pl.ds(i * L, L)] = jnp.zeros((L,), jnp.float32)

    pltpu.sync_copy(zb_v, shared.at[pl.ds(s * ROWS_PER_TILE, ROWS_PER_TILE)])
    plsc.subcore_barrier()
    gbase = (c * NS + s) * G_E
    pltpu.sync_copy(dst2_hbm.at[pl.ds(gbase, G_E)], idx_v)

    @pl.loop(0, G_E)
    def _(j):
        pltpu.sync_copy(ones_v, shared.at[idx_v.at[j]], add=True)

    plsc.subcore_barrier()
    sl = pl.ds(s * ROWS_PER_TILE, ROWS_PER_TILE)
    pltpu.sync_copy(shared.at[sl], deg_hbm.at[c, sl])


def _deg(dst2):
    f = functools.partial(
        pl.kernel,
        out_type=jax.ShapeDtypeStruct((NC, NP), jnp.float32),
        mesh=_mesh(),
        scratch_types=[
            pltpu.VMEM((G_E, 128), jnp.int32),
            pltpu.VMEM((128,), jnp.float32),
            pltpu.VMEM((ROWS_PER_TILE,), jnp.float32),
            pltpu.VMEM_SHARED((NP,), jnp.float32),
        ],
    )(_deg_body)
    return f(dst2)


_HALF = G_E // 2


def _msg_body(g_hbm, src2_hbm, dst2_hbm, acc_hbm, isrc_v, idst_v, rows_a, rows_b,
              shared, sem_ga, sem_gb, sem_sa, sem_sb):
    c = lax.axis_index("c")
    s = lax.axis_index("s")

    @pl.loop(0, 128)
    def _(r):
        @pl.loop(0, D // L)
        def _(k):
            rows_a[r, pl.ds(k * L, L)] = jnp.zeros((L,), jnp.float32)

    @pl.loop(0, ROWS_PER_TILE // 128)
    def _(i):
        pltpu.sync_copy(rows_a, shared.at[pl.ds(s * ROWS_PER_TILE + i * 128, 128)])

    plsc.subcore_barrier()
    gbase = (c * NS + s) * G_E

    _NSPL = 4
    _SR = 128 // _NSPL

    def start_gather(j, buf, sem):
        for hh in range(_NSPL):
            pltpu.async_copy(
                g_hbm.at[isrc_v.at[j, pl.ds(hh * _SR, _SR)]],
                buf.at[pl.ds(hh * _SR, _SR)],
                sem,
            )

    def wait_gather(buf, sem):
        for hh in range(_NSPL):
            pltpu.make_async_copy(
                g_hbm.at[isrc_v.at[0, pl.ds(hh * _SR, _SR)]],
                buf.at[pl.ds(hh * _SR, _SR)],
                sem,
            ).wait()

    def wait_scatter(buf, sem):
        pltpu.make_async_copy(buf, shared.at[idst_v.at[0]], sem).wait()

    for half in range(2):
        base = gbase + half * _HALF
        pltpu.sync_copy(src2_hbm.at[pl.ds(base, _HALF)], isrc_v)
        pltpu.sync_copy(dst2_hbm.at[pl.ds(base, _HALF)], idst_v)
        start_gather(0, rows_a, sem_ga)

        @pl.loop(0, _HALF // 2)
        def _(i):
            j = 2 * i
            wait_gather(rows_a, sem_ga)

            @pl.when(i >= 1)
            def _():
                wait_scatter(rows_b, sem_sb)

            start_gather(j + 1, rows_b, sem_gb)
            pltpu.async_copy(rows_a, shared.at[idst_v.at[j]], sem_sa, add=True)
            wait_gather(rows_b, sem_gb)
            wait_scatter(rows_a, sem_sa)

            @pl.when(j + 2 < _HALF)
            def _():
                start_gather(j + 2, rows_a, sem_ga)

            pltpu.async_copy(rows_b, shared.at[idst_v.at[j + 1]], sem_sb, add=True)

        wait_scatter(rows_b, sem_sb)

    plsc.subcore_barrier()

    @pl.loop(0, ROWS_PER_TILE // 128)
    def _(i):
        sl = pl.ds(s * ROWS_PER_TILE + i * 128, 128)
        pltpu.sync_copy(shared.at[sl], acc_hbm.at[c, sl])


def _msg(g, src2, dst2):
    f = functools.partial(
        pl.kernel,
        out_type=jax.ShapeDtypeStruct((NC, NP, D), jnp.float32),
        mesh=_mesh(),
        scratch_types=[
            pltpu.VMEM((_HALF, 128), jnp.int32),
            pltpu.VMEM((_HALF, 128), jnp.int32),
            pltpu.VMEM((128, D), jnp.float32),
            pltpu.VMEM((128, D), jnp.float32),
            pltpu.VMEM_SHARED((NP, D), jnp.float32),
            pltpu.SemaphoreType.DMA,
            pltpu.SemaphoreType.DMA,
            pltpu.SemaphoreType.DMA,
            pltpu.SemaphoreType.DMA,
        ],
    )(_msg_body)
    return f(g, src2, dst2)


def _score_body(p_hbm, q_hbm, a2_hbm, b2_hbm, adj_hbm, p_v, q_v, ia_v, ib_v, out_v):
    c = lax.axis_index("c")
    s = lax.axis_index("s")
    pltpu.sync_copy(p_hbm, p_v)
    pltpu.sync_copy(q_hbm, q_v)
    gbase = (c * NS + s) * G_S
    pltpu.sync_copy(a2_hbm.at[pl.ds(gbase, G_S)], ia_v)
    pltpu.sync_copy(b2_hbm.at[pl.ds(gbase, G_S)], ib_v)

    @pl.loop(0, G_S)
    def _(j):
        for k in range(128 // L):
            sl = pl.ds(k * L, L)
            va = plsc.load_gather(p_v, [ia_v[j, sl]])
            vb = plsc.load_gather(q_v, [ib_v[j, sl]])
            out_v[j, sl] = 1.0 / (1.0 + jnp.exp(-(va + vb)))

    pltpu.sync_copy(out_v, adj_hbm.at[pl.ds(gbase, G_S)])


def _score(p, q, a2, b2):
    cp = pltpu.CompilerParams()
    if "needs_layout_passes" in pltpu.CompilerParams.__dataclass_fields__:
        cp = dataclasses.replace(cp, needs_layout_passes=False)
    f = functools.partial(
        pl.kernel,
        out_type=jax.ShapeDtypeStruct((S_PAD // 128, 128), jnp.float32),
        mesh=_mesh(),
        compiler_params=cp,
        scratch_types=[
            pltpu.VMEM((N,), jnp.float32),
            pltpu.VMEM((N,), jnp.float32),
            pltpu.VMEM((G_S, 128), jnp.int32),
            pltpu.VMEM((G_S, 128), jnp.int32),
            pltpu.VMEM((G_S, 128), jnp.float32),
        ],
    )(_score_body)
    return f(p, q, a2, b2)


_BLK = 2000


def _mm_body(x_ref, w_ref, o_ref):
    o_ref[...] = jnp.dot(x_ref[...], w_ref[...], preferred_element_type=jnp.float32)


def _mm(x, w):
    return pl.pallas_call(
        _mm_body,
        grid=(N // _BLK,),
        in_specs=[
            pl.BlockSpec((_BLK, D), lambda i: (i, 0)),
            pl.BlockSpec((D, D), lambda i: (0, 0)),
        ],
        out_specs=pl.BlockSpec((_BLK, D), lambda i: (i, 0)),
        out_shape=jax.ShapeDtypeStruct((N, D), jnp.float32),
    )(x, w)


def _gscale_body(h_ref, d0_ref, d1_ref, g_ref, dinv_ref):
    deg = d0_ref[...] + d1_ref[...] + 1.0
    dinv = lax.rsqrt(deg)
    dinv_ref[...] = dinv
    g_ref[...] = h_ref[...] * dinv


def _gscale(h, d0, d1):
    return pl.pallas_call(
        _gscale_body,
        grid=(N // _BLK,),
        in_specs=[
            pl.BlockSpec((_BLK, D), lambda i: (i, 0)),
            pl.BlockSpec((_BLK, 1), lambda i: (i, 0)),
            pl.BlockSpec((_BLK, 1), lambda i: (i, 0)),
        ],
        out_specs=[
            pl.BlockSpec((_BLK, D), lambda i: (i, 0)),
            pl.BlockSpec((_BLK, 1), lambda i: (i, 0)),
        ],
        out_shape=[
            jax.ShapeDtypeStruct((N, D), jnp.float32),
            jax.ShapeDtypeStruct((N, 1), jnp.float32),
        ],
    )(h, d0, d1)


def _head_body(a0_ref, a1_ref, h_ref, dinv_ref, b1_ref, wm1_ref, bm1_ref,
               wm2_ref, bm2_ref, wa1_ref, wa2_ref, ba_ref,
               f_ref, p_ref, q_ref):
    dinv = dinv_ref[...]
    out = dinv * (a0_ref[...] + a1_ref[...]) + (dinv * dinv) * h_ref[...] + b1_ref[...]
    t = jnp.maximum(
        jnp.dot(out, wm1_ref[...], preferred_element_type=jnp.float32) + bm1_ref[...],
        0.0,
    )
    f_ref[...] = jax.nn.sigmoid(
        jnp.dot(t, wm2_ref[...], preferred_element_type=jnp.float32) + bm2_ref[...]
    )
    p_ref[...] = jnp.dot(out, wa1_ref[...], preferred_element_type=jnp.float32) + ba_ref[0, 0]
    q_ref[...] = jnp.dot(out, wa2_ref[...], preferred_element_type=jnp.float32)


def _head(a0, a1, h, dinv, b1, wm1, bm1, wm2, bm2, wa1, wa2, ba):
    full = lambda shape: pl.BlockSpec(shape, lambda i: tuple(0 for _ in shape))
    return pl.pallas_call(
        _head_body,
        grid=(N // _BLK,),
        in_specs=[
            pl.BlockSpec((_BLK, D), lambda i: (i, 0)),
            pl.BlockSpec((_BLK, D), lambda i: (i, 0)),
            pl.BlockSpec((_BLK, D), lambda i: (i, 0)),
            pl.BlockSpec((_BLK, 1), lambda i: (i, 0)),
            full((1, D)),
            full((D, D)),
            full((1, D)),
            full((D, D)),
            full((1, D)),
            full((D, 1)),
            full((D, 1)),
            full((1, 1)),
        ],
        out_specs=[
            pl.BlockSpec((_BLK, D), lambda i: (i, 0)),
            pl.BlockSpec((_BLK, 1), lambda i: (i, 0)),
            pl.BlockSpec((_BLK, 1), lambda i: (i, 0)),
        ],
        out_shape=[
            jax.ShapeDtypeStruct((N, D), jnp.float32),
            jax.ShapeDtypeStruct((N, 1), jnp.float32),
            jax.ShapeDtypeStruct((N, 1), jnp.float32),
        ],
    )(a0, a1, h, dinv, b1, wm1, bm1, wm2, bm2, wa1, wa2, ba)


def kernel(x, edge_index, k_hop_edge_index, neg_adj, W1, b1, Wm1, bm1, Wm2, bm2, Wa, ba):
    src = edge_index[0]
    dst = edge_index[1]
    epad = E_PAD - E
    pad_src = jnp.arange(epad, dtype=jnp.int32) % N
    pad_dst = N + jnp.arange(epad, dtype=jnp.int32) % (NP - N)
    src2 = jnp.concatenate([src, pad_src]).reshape(-1, 128)
    dst2 = jnp.concatenate([dst, pad_dst]).reshape(-1, 128)
    spad = S_PAD - S
    pad_ab = jnp.arange(spad, dtype=jnp.int32) % N
    a2 = jnp.concatenate(
        [k_hop_edge_index[0], neg_adj[:, 0], pad_ab]
    ).reshape(-1, 128)
    b2 = jnp.concatenate(
        [k_hop_edge_index[1], neg_adj[:, 1], pad_ab]
    ).reshape(-1, 128)

    h = _mm(x, W1)
    deg_parts = _deg(dst2)
    d0 = deg_parts[0].reshape(NP, 1)
    d1 = deg_parts[1].reshape(NP, 1)
    g, dinv = _gscale(h, d0[:N], d1[:N])
    acc = _msg(g, src2, dst2)
    f, p, q = _head(
        acc[0], acc[1], h, dinv,
        b1.reshape(1, D), Wm1, bm1.reshape(1, D), Wm2, bm2.reshape(1, D),
        Wa[:D], Wa[D:], ba.reshape(1, 1),
    )
    adj2 = _score(p.reshape(-1), q.reshape(-1), a2, b2)
    adj_out = adj2.reshape(-1)[:S]
    return f, adj_out


# speedup vs baseline: 52.4925x; 1.0156x over previous
import dataclasses
import functools

import jax
import jax.numpy as jnp
from jax import lax
from jax.experimental import pallas as pl
from jax.experimental.pallas import tpu as pltpu
from jax.experimental.pallas import tpu_sc as plsc

N = 10000
D = 128
NP = 10240
DUMP = 10008
NC, NS, L = 2, 16, 16
NW = NC * NS

E = 320000
E_PAD = 327680
G_E = E_PAD // (NW * 128)

S = 960000
S_PAD = 983040
G_S = S_PAD // (NW * 128)

ROWS_PER_TILE = NP // NS


def _mesh():
    return plsc.VectorSubcoreMesh(core_axis_name="c", subcore_axis_name="s")


def _deg_body(dst2_hbm, deg_hbm, idx_v, ones_v, zb_v, shared):
    c = lax.axis_index("c")
    s = lax.axis_index("s")

    @pl.loop(0, 8)
    def _(i):
        ones_v[pl.ds(i * L, L)] = jnp.ones((L,), jnp.float32)

    @pl.loop(0, ROWS_PER_TILE // L)
    def _(i):
        zb_v[pl.ds(i * L, L)] = jnp.zeros((L,), jnp.float32)

    pltpu.sync_copy(zb_v, shared.at[pl.ds(s * ROWS_PER_TILE, ROWS_PER_TILE)])
    plsc.subcore_barrier()
    gbase = (c * NS + s) * G_E
    pltpu.sync_copy(dst2_hbm.at[pl.ds(gbase, G_E)], idx_v)

    @pl.loop(0, G_E)
    def _(j):
        pltpu.sync_copy(ones_v, shared.at[idx_v.at[j]], add=True)

    plsc.subcore_barrier()
    sl = pl.ds(s * ROWS_PER_TILE, ROWS_PER_TILE)
    pltpu.sync_copy(shared.at[sl], deg_hbm.at[c, sl])


def _deg(dst2):
    f = functools.partial(
        pl.kernel,
        out_type=jax.ShapeDtypeStruct((NC, NP), jnp.float32),
        mesh=_mesh(),
        scratch_types=[
            pltpu.VMEM((G_E, 128), jnp.int32),
            pltpu.VMEM((128,), jnp.float32),
            pltpu.VMEM((ROWS_PER_TILE,), jnp.float32),
            pltpu.VMEM_SHARED((NP,), jnp.float32),
        ],
    )(_deg_body)
    return f(dst2)


_HALF = G_E // 2


def _msg_body(g_hbm, src2_hbm, dst2_hbm, acc_hbm, isrc_v, idst_v, rows_a, rows_b,
              shared, sem_ga, sem_gb, sem_sa, sem_sb):
    c = lax.axis_index("c")
    s = lax.axis_index("s")

    @pl.loop(0, 128)
    def _(r):
        @pl.loop(0, D // L)
        def _(k):
            rows_a[r, pl.ds(k * L, L)] = jnp.zeros((L,), jnp.float32)

    @pl.loop(0, ROWS_PER_TILE // 128)
    def _(i):
        pltpu.sync_copy(rows_a, shared.at[pl.ds(s * ROWS_PER_TILE + i * 128, 128)])

    plsc.subcore_barrier()
    gbase = (c * NS + s) * G_E

    _NSPL = 4
    _SR = 128 // _NSPL

    def start_gather(j, buf, sem):
        for hh in range(_NSPL):
            pltpu.async_copy(
                g_hbm.at[isrc_v.at[j, pl.ds(hh * _SR, _SR)]],
                buf.at[pl.ds(hh * _SR, _SR)],
                sem,
            )

    def wait_gather(buf, sem):
        for hh in range(_NSPL):
            pltpu.make_async_copy(
                g_hbm.at[isrc_v.at[0, pl.ds(hh * _SR, _SR)]],
                buf.at[pl.ds(hh * _SR, _SR)],
                sem,
            ).wait()

    def wait_scatter(buf, sem):
        pltpu.make_async_copy(buf, shared.at[idst_v.at[0]], sem).wait()

    for half in range(2):
        base = gbase + half * _HALF
        pltpu.sync_copy(src2_hbm.at[pl.ds(base, _HALF)], isrc_v)
        pltpu.sync_copy(dst2_hbm.at[pl.ds(base, _HALF)], idst_v)
        start_gather(0, rows_a, sem_ga)

        @pl.loop(0, _HALF // 2)
        def _(i):
            j = 2 * i
            wait_gather(rows_a, sem_ga)

            @pl.when(i >= 1)
            def _():
                wait_scatter(rows_b, sem_sb)

            start_gather(j + 1, rows_b, sem_gb)
            pltpu.async_copy(rows_a, shared.at[idst_v.at[j]], sem_sa, add=True)
            wait_gather(rows_b, sem_gb)
            wait_scatter(rows_a, sem_sa)

            @pl.when(j + 2 < _HALF)
            def _():
                start_gather(j + 2, rows_a, sem_ga)

            pltpu.async_copy(rows_b, shared.at[idst_v.at[j + 1]], sem_sb, add=True)

        wait_scatter(rows_b, sem_sb)

    plsc.subcore_barrier()

    @pl.loop(0, ROWS_PER_TILE // 128)
    def _(i):
        sl = pl.ds(s * ROWS_PER_TILE + i * 128, 128)
        pltpu.sync_copy(shared.at[sl], acc_hbm.at[c, sl])


def _msg(g, src2, dst2):
    f = functools.partial(
        pl.kernel,
        out_type=jax.ShapeDtypeStruct((NC, NP, D), jnp.float32),
        mesh=_mesh(),
        scratch_types=[
            pltpu.VMEM((_HALF, 128), jnp.int32),
            pltpu.VMEM((_HALF, 128), jnp.int32),
            pltpu.VMEM((128, D), jnp.float32),
            pltpu.VMEM((128, D), jnp.float32),
            pltpu.VMEM_SHARED((NP, D), jnp.float32),
            pltpu.SemaphoreType.DMA,
            pltpu.SemaphoreType.DMA,
            pltpu.SemaphoreType.DMA,
            pltpu.SemaphoreType.DMA,
        ],
    )(_msg_body)
    return f(g, src2, dst2)


_SCH = 48
_SNC = G_S // _SCH


def _score_body(p_hbm, q_hbm, a2_hbm, b2_hbm, adj_hbm,
                p_v, q_v, ia0, ib0, ia1, ib1, out0, out1, semp, semi, semo):
    c = lax.axis_index("c")
    s = lax.axis_index("s")
    gbase = (c * NS + s) * G_S
    ia = (ia0, ia1)
    ib = (ib0, ib1)
    ob = (out0, out1)
    pltpu.async_copy(p_hbm, p_v, semp)
    pltpu.async_copy(q_hbm, q_v, semp)
    pltpu.async_copy(a2_hbm.at[pl.ds(gbase, _SCH)], ia0, semi)
    pltpu.async_copy(b2_hbm.at[pl.ds(gbase, _SCH)], ib0, semi)
    pltpu.make_async_copy(p_hbm, p_v, semp).wait()
    pltpu.make_async_copy(q_hbm, q_v, semp).wait()

    for ci in range(_SNC):
        cur = ci % 2
        nxt = (ci + 1) % 2
        iac, ibc, obc = ia[cur], ib[cur], ob[cur]
        pltpu.make_async_copy(a2_hbm.at[pl.ds(gbase, _SCH)], iac, semi).wait()
        pltpu.make_async_copy(b2_hbm.at[pl.ds(gbase, _SCH)], ibc, semi).wait()
        if ci + 1 < _SNC:
            nbase = gbase + (ci + 1) * _SCH
            pltpu.async_copy(a2_hbm.at[pl.ds(nbase, _SCH)], ia[nxt], semi)
            pltpu.async_copy(b2_hbm.at[pl.ds(nbase, _SCH)], ib[nxt], semi)
        if ci >= 2:
            pltpu.make_async_copy(obc, adj_hbm.at[pl.ds(gbase, _SCH)], semo).wait()

        @pl.loop(0, _SCH)
        def _(j):
            for k in range(128 // L):
                sl = pl.ds(k * L, L)
                va = plsc.load_gather(p_v, [iac[j, sl]])
                vb = plsc.load_gather(q_v, [ibc[j, sl]])
                obc[j, sl] = 1.0 / (1.0 + jnp.exp(-(va + vb)))

        pltpu.async_copy(obc, adj_hbm.at[pl.ds(gbase + ci * _SCH, _SCH)], semo)

    pltpu.make_async_copy(out0, adj_hbm.at[pl.ds(gbase, _SCH)], semo).wait()
    pltpu.make_async_copy(out1, adj_hbm.at[pl.ds(gbase, _SCH)], semo).wait()


def _score(p, q, a2, b2):
    cp = pltpu.CompilerParams()
    if "needs_layout_passes" in pltpu.CompilerParams.__dataclass_fields__:
        cp = dataclasses.replace(cp, needs_layout_passes=False)
    f = functools.partial(
        pl.kernel,
        out_type=jax.ShapeDtypeStruct((S_PAD // 128, 128), jnp.float32),
        mesh=_mesh(),
        compiler_params=cp,
        scratch_types=[
            pltpu.VMEM((N,), jnp.float32),
            pltpu.VMEM((N,), jnp.float32),
            pltpu.VMEM((_SCH, 128), jnp.int32),
            pltpu.VMEM((_SCH, 128), jnp.int32),
            pltpu.VMEM((_SCH, 128), jnp.int32),
            pltpu.VMEM((_SCH, 128), jnp.int32),
            pltpu.VMEM((_SCH, 128), jnp.float32),
            pltpu.VMEM((_SCH, 128), jnp.float32),
            pltpu.SemaphoreType.DMA,
            pltpu.SemaphoreType.DMA,
            pltpu.SemaphoreType.DMA,
        ],
    )(_score_body)
    return f(p, q, a2, b2)


_BLK = 2000


def _mm_body(x_ref, w_ref, o_ref):
    o_ref[...] = jnp.dot(x_ref[...], w_ref[...], preferred_element_type=jnp.float32)


def _mm(x, w):
    return pl.pallas_call(
        _mm_body,
        grid=(N // _BLK,),
        in_specs=[
            pl.BlockSpec((_BLK, D), lambda i: (i, 0)),
            pl.BlockSpec((D, D), lambda i: (0, 0)),
        ],
        out_specs=pl.BlockSpec((_BLK, D), lambda i: (i, 0)),
        out_shape=jax.ShapeDtypeStruct((N, D), jnp.float32),
    )(x, w)


def _gscale_body(h_ref, d0_ref, d1_ref, g_ref, dinv_ref):
    deg = d0_ref[...] + d1_ref[...] + 1.0
    dinv = lax.rsqrt(deg)
    dinv_ref[...] = dinv
    g_ref[...] = h_ref[...] * dinv


def _gscale(h, d0, d1):
    return pl.pallas_call(
        _gscale_body,
        grid=(N // _BLK,),
        in_specs=[
            pl.BlockSpec((_BLK, D), lambda i: (i, 0)),
            pl.BlockSpec((_BLK, 1), lambda i: (i, 0)),
            pl.BlockSpec((_BLK, 1), lambda i: (i, 0)),
        ],
        out_specs=[
            pl.BlockSpec((_BLK, D), lambda i: (i, 0)),
            pl.BlockSpec((_BLK, 1), lambda i: (i, 0)),
        ],
        out_shape=[
            jax.ShapeDtypeStruct((N, D), jnp.float32),
            jax.ShapeDtypeStruct((N, 1), jnp.float32),
        ],
    )(h, d0, d1)


def _head_body(a0_ref, a1_ref, h_ref, dinv_ref, b1_ref, wm1_ref, bm1_ref,
               wm2_ref, bm2_ref, wa1_ref, wa2_ref, ba_ref,
               f_ref, p_ref, q_ref):
    dinv = dinv_ref[...]
    out = dinv * (a0_ref[...] + a1_ref[...]) + (dinv * dinv) * h_ref[...] + b1_ref[...]
    t = jnp.maximum(
        jnp.dot(out, wm1_ref[...], preferred_element_type=jnp.float32) + bm1_ref[...],
        0.0,
    )
    f_ref[...] = jax.nn.sigmoid(
        jnp.dot(t, wm2_ref[...], preferred_element_type=jnp.float32) + bm2_ref[...]
    )
    p_ref[...] = jnp.dot(out, wa1_ref[...], preferred_element_type=jnp.float32) + ba_ref[0, 0]
    q_ref[...] = jnp.dot(out, wa2_ref[...], preferred_element_type=jnp.float32)


def _head(a0, a1, h, dinv, b1, wm1, bm1, wm2, bm2, wa1, wa2, ba):
    full = lambda shape: pl.BlockSpec(shape, lambda i: tuple(0 for _ in shape))
    return pl.pallas_call(
        _head_body,
        grid=(N // _BLK,),
        in_specs=[
            pl.BlockSpec((_BLK, D), lambda i: (i, 0)),
            pl.BlockSpec((_BLK, D), lambda i: (i, 0)),
            pl.BlockSpec((_BLK, D), lambda i: (i, 0)),
            pl.BlockSpec((_BLK, 1), lambda i: (i, 0)),
            full((1, D)),
            full((D, D)),
            full((1, D)),
            full((D, D)),
            full((1, D)),
            full((D, 1)),
            full((D, 1)),
            full((1, 1)),
        ],
        out_specs=[
            pl.BlockSpec((_BLK, D), lambda i: (i, 0)),
            pl.BlockSpec((_BLK, 1), lambda i: (i, 0)),
            pl.BlockSpec((_BLK, 1), lambda i: (i, 0)),
        ],
        out_shape=[
            jax.ShapeDtypeStruct((N, D), jnp.float32),
            jax.ShapeDtypeStruct((N, 1), jnp.float32),
            jax.ShapeDtypeStruct((N, 1), jnp.float32),
        ],
    )(a0, a1, h, dinv, b1, wm1, bm1, wm2, bm2, wa1, wa2, ba)


def kernel(x, edge_index, k_hop_edge_index, neg_adj, W1, b1, Wm1, bm1, Wm2, bm2, Wa, ba):
    src = edge_index[0]
    dst = edge_index[1]
    epad = E_PAD - E
    pad_src = jnp.arange(epad, dtype=jnp.int32) % N
    pad_dst = N + jnp.arange(epad, dtype=jnp.int32) % (NP - N)
    src2 = jnp.concatenate([src, pad_src]).reshape(-1, 128)
    dst2 = jnp.concatenate([dst, pad_dst]).reshape(-1, 128)
    spad = S_PAD - S
    pad_ab = jnp.arange(spad, dtype=jnp.int32) % N
    a2 = jnp.concatenate(
        [k_hop_edge_index[0], neg_adj[:, 0], pad_ab]
    ).reshape(-1, 128)
    b2 = jnp.concatenate(
        [k_hop_edge_index[1], neg_adj[:, 1], pad_ab]
    ).reshape(-1, 128)

    h = _mm(x, W1)
    deg_parts = _deg(dst2)
    d0 = deg_parts[0].reshape(NP, 1)
    d1 = deg_parts[1].reshape(NP, 1)
    g, dinv = _gscale(h, d0[:N], d1[:N])
    acc = _msg(g, src2, dst2)
    f, p, q = _head(
        acc[0], acc[1], h, dinv,
        b1.reshape(1, D), Wm1, bm1.reshape(1, D), Wm2, bm2.reshape(1, D),
        Wa[:D], Wa[D:], ba.reshape(1, 1),
    )
    adj2 = _score(p.reshape(-1), q.reshape(-1), a2, b2)
    adj_out = adj2.reshape(-1)[:S]
    return f, adj_out


# speedup vs baseline: 56.8899x; 1.0838x over previous
import dataclasses
import functools

import jax
import jax.numpy as jnp
from jax import lax
from jax.experimental import pallas as pl
from jax.experimental.pallas import tpu as pltpu
from jax.experimental.pallas import tpu_sc as plsc

N = 10000
D = 128
NP = 10240
DUMP = 10008
NC, NS, L = 2, 16, 16
NW = NC * NS

E = 320000
E_PAD = 327680
G_E = E_PAD // (NW * 128)

S = 960000
S_PAD = 983040
G_S = S_PAD // (NW * 128)

ROWS_PER_TILE = NP // NS


def _mesh():
    return plsc.VectorSubcoreMesh(core_axis_name="c", subcore_axis_name="s")


def _deg_body(dst2_hbm, deg_hbm, idx_v, ones_v, zb_v, shared):
    c = lax.axis_index("c")
    s = lax.axis_index("s")

    @pl.loop(0, 8)
    def _(i):
        ones_v[pl.ds(i * L, L)] = jnp.ones((L,), jnp.float32)

    @pl.loop(0, ROWS_PER_TILE // L)
    def _(i):
        zb_v[pl.ds(i * L, L)] = jnp.zeros((L,), jnp.float32)

    pltpu.sync_copy(zb_v, shared.at[pl.ds(s * ROWS_PER_TILE, ROWS_PER_TILE)])
    plsc.subcore_barrier()
    gbase = (c * NS + s) * G_E
    pltpu.sync_copy(dst2_hbm.at[pl.ds(gbase, G_E)], idx_v)

    @pl.loop(0, G_E)
    def _(j):
        pltpu.sync_copy(ones_v, shared.at[idx_v.at[j]], add=True)

    plsc.subcore_barrier()
    sl = pl.ds(s * ROWS_PER_TILE, ROWS_PER_TILE)
    pltpu.sync_copy(shared.at[sl], deg_hbm.at[c, sl])


def _deg(dst2):
    f = functools.partial(
        pl.kernel,
        out_type=jax.ShapeDtypeStruct((NC, NP), jnp.float32),
        mesh=_mesh(),
        scratch_types=[
            pltpu.VMEM((G_E, 128), jnp.int32),
            pltpu.VMEM((128,), jnp.float32),
            pltpu.VMEM((ROWS_PER_TILE,), jnp.float32),
            pltpu.VMEM_SHARED((NP,), jnp.float32),
        ],
    )(_deg_body)
    return f(dst2)


_HALF = G_E // 2


def _msg_body(g_hbm, src2_hbm, dst2_hbm, acc_hbm, isrc_v, idst_v, rows_a, rows_b,
              shared, sem_ga, sem_gb, sem_sa, sem_sb):
    c = lax.axis_index("c")
    s = lax.axis_index("s")

    @pl.loop(0, 128)
    def _(r):
        @pl.loop(0, D // L)
        def _(k):
            rows_a[r, pl.ds(k * L, L)] = jnp.zeros((L,), jnp.float32)

    @pl.loop(0, ROWS_PER_TILE // 128)
    def _(i):
        pltpu.sync_copy(rows_a, shared.at[pl.ds(s * ROWS_PER_TILE + i * 128, 128)])

    plsc.subcore_barrier()
    gbase = (c * NS + s) * G_E

    _NSPL = 4
    _SR = 128 // _NSPL

    def start_gather(j, buf, sem):
        for hh in range(_NSPL):
            pltpu.async_copy(
                g_hbm.at[isrc_v.at[j, pl.ds(hh * _SR, _SR)]],
                buf.at[pl.ds(hh * _SR, _SR)],
                sem,
            )

    def wait_gather(buf, sem):
        for hh in range(_NSPL):
            pltpu.make_async_copy(
                g_hbm.at[isrc_v.at[0, pl.ds(hh * _SR, _SR)]],
                buf.at[pl.ds(hh * _SR, _SR)],
                sem,
            ).wait()

    def wait_scatter(buf, sem):
        pltpu.make_async_copy(buf, shared.at[idst_v.at[0]], sem).wait()

    for half in range(2):
        base = gbase + half * _HALF
        pltpu.sync_copy(src2_hbm.at[pl.ds(base, _HALF)], isrc_v)
        pltpu.sync_copy(dst2_hbm.at[pl.ds(base, _HALF)], idst_v)
        start_gather(0, rows_a, sem_ga)

        @pl.loop(0, _HALF // 2)
        def _(i):
            j = 2 * i
            wait_gather(rows_a, sem_ga)

            @pl.when(i >= 1)
            def _():
                wait_scatter(rows_b, sem_sb)

            start_gather(j + 1, rows_b, sem_gb)
            pltpu.async_copy(rows_a, shared.at[idst_v.at[j]], sem_sa, add=True)
            wait_gather(rows_b, sem_gb)
            wait_scatter(rows_a, sem_sa)

            @pl.when(j + 2 < _HALF)
            def _():
                start_gather(j + 2, rows_a, sem_ga)

            pltpu.async_copy(rows_b, shared.at[idst_v.at[j + 1]], sem_sb, add=True)

        wait_scatter(rows_b, sem_sb)

    plsc.subcore_barrier()

    @pl.loop(0, ROWS_PER_TILE // 128)
    def _(i):
        sl = pl.ds(s * ROWS_PER_TILE + i * 128, 128)
        pltpu.sync_copy(shared.at[sl], acc_hbm.at[c, sl])


def _msg(g, src2, dst2):
    f = functools.partial(
        pl.kernel,
        out_type=jax.ShapeDtypeStruct((NC, NP, D), jnp.float32),
        mesh=_mesh(),
        scratch_types=[
            pltpu.VMEM((_HALF, 128), jnp.int32),
            pltpu.VMEM((_HALF, 128), jnp.int32),
            pltpu.VMEM((128, D), jnp.float32),
            pltpu.VMEM((128, D), jnp.float32),
            pltpu.VMEM_SHARED((NP, D), jnp.float32),
            pltpu.SemaphoreType.DMA,
            pltpu.SemaphoreType.DMA,
            pltpu.SemaphoreType.DMA,
            pltpu.SemaphoreType.DMA,
        ],
    )(_msg_body)
    return f(g, src2, dst2)


_SCH = 48
_SNC = G_S // _SCH


def _score_body(p_hbm, q_hbm, a2_hbm, b2_hbm, adj_hbm,
                p_v, q_v, ia0, ib0, ia1, ib1, out0, out1, semp, semi, semo):
    c = lax.axis_index("c")
    s = lax.axis_index("s")
    gbase = (c * NS + s) * G_S
    ia = (ia0, ia1)
    ib = (ib0, ib1)
    ob = (out0, out1)
    pltpu.async_copy(p_hbm, p_v, semp)
    pltpu.async_copy(q_hbm, q_v, semp)
    pltpu.async_copy(a2_hbm.at[pl.ds(gbase, _SCH)], ia0, semi)
    pltpu.async_copy(b2_hbm.at[pl.ds(gbase, _SCH)], ib0, semi)
    pltpu.make_async_copy(p_hbm, p_v, semp).wait()
    pltpu.make_async_copy(q_hbm, q_v, semp).wait()

    for ci in range(_SNC):
        cur = ci % 2
        nxt = (ci + 1) % 2
        iac, ibc, obc = ia[cur], ib[cur], ob[cur]
        pltpu.make_async_copy(a2_hbm.at[pl.ds(gbase, _SCH)], iac, semi).wait()
        pltpu.make_async_copy(b2_hbm.at[pl.ds(gbase, _SCH)], ibc, semi).wait()
        if ci + 1 < _SNC:
            nbase = gbase + (ci + 1) * _SCH
            pltpu.async_copy(a2_hbm.at[pl.ds(nbase, _SCH)], ia[nxt], semi)
            pltpu.async_copy(b2_hbm.at[pl.ds(nbase, _SCH)], ib[nxt], semi)
        if ci >= 2:
            pltpu.make_async_copy(obc, adj_hbm.at[pl.ds(gbase, _SCH)], semo).wait()

        @pl.loop(0, _SCH)
        def _(j):
            for k in range(128 // L):
                sl = pl.ds(k * L, L)
                va = plsc.load_gather(p_v, [iac[j, sl]])
                vb = plsc.load_gather(q_v, [ibc[j, sl]])
                obc[j, sl] = va + vb

        pltpu.async_copy(obc, adj_hbm.at[pl.ds(gbase + ci * _SCH, _SCH)], semo)

    pltpu.make_async_copy(out0, adj_hbm.at[pl.ds(gbase, _SCH)], semo).wait()
    pltpu.make_async_copy(out1, adj_hbm.at[pl.ds(gbase, _SCH)], semo).wait()


def _score(p, q, a2, b2):
    cp = pltpu.CompilerParams()
    if "needs_layout_passes" in pltpu.CompilerParams.__dataclass_fields__:
        cp = dataclasses.replace(cp, needs_layout_passes=False)
    f = functools.partial(
        pl.kernel,
        out_type=jax.ShapeDtypeStruct((S_PAD // 128, 128), jnp.float32),
        mesh=_mesh(),
        compiler_params=cp,
        scratch_types=[
            pltpu.VMEM((N,), jnp.float32),
            pltpu.VMEM((N,), jnp.float32),
            pltpu.VMEM((_SCH, 128), jnp.int32),
            pltpu.VMEM((_SCH, 128), jnp.int32),
            pltpu.VMEM((_SCH, 128), jnp.int32),
            pltpu.VMEM((_SCH, 128), jnp.int32),
            pltpu.VMEM((_SCH, 128), jnp.float32),
            pltpu.VMEM((_SCH, 128), jnp.float32),
            pltpu.SemaphoreType.DMA,
            pltpu.SemaphoreType.DMA,
            pltpu.SemaphoreType.DMA,
        ],
    )(_score_body)
    return f(p, q, a2, b2)


_BLK = 2000
_SROWS = S // 128


def _sig_body(s_ref, o_ref):
    o_ref[...] = jax.nn.sigmoid(s_ref[...])


def _sig(sraw):
    rows = S_PAD // 128
    return pl.pallas_call(
        _sig_body,
        grid=(5,),
        in_specs=[pl.BlockSpec((rows // 5, 128), lambda i: (i, 0))],
        out_specs=pl.BlockSpec((rows // 5, 128), lambda i: (i, 0)),
        out_shape=jax.ShapeDtypeStruct((rows, 128), jnp.float32),
    )(sraw)


def _mm_body(x_ref, w_ref, o_ref):
    o_ref[...] = jnp.dot(x_ref[...], w_ref[...], preferred_element_type=jnp.float32)


def _mm(x, w):
    return pl.pallas_call(
        _mm_body,
        grid=(N // _BLK,),
        in_specs=[
            pl.BlockSpec((_BLK, D), lambda i: (i, 0)),
            pl.BlockSpec((D, D), lambda i: (0, 0)),
        ],
        out_specs=pl.BlockSpec((_BLK, D), lambda i: (i, 0)),
        out_shape=jax.ShapeDtypeStruct((N, D), jnp.float32),
    )(x, w)


def _gscale_body(h_ref, d0_ref, d1_ref, g_ref, dinv_ref):
    deg = d0_ref[...] + d1_ref[...] + 1.0
    dinv = lax.rsqrt(deg)
    dinv_ref[...] = dinv
    g_ref[...] = h_ref[...] * dinv


def _gscale(h, d0, d1):
    return pl.pallas_call(
        _gscale_body,
        grid=(N // _BLK,),
        in_specs=[
            pl.BlockSpec((_BLK, D), lambda i: (i, 0)),
            pl.BlockSpec((_BLK, 1), lambda i: (i, 0)),
            pl.BlockSpec((_BLK, 1), lambda i: (i, 0)),
        ],
        out_specs=[
            pl.BlockSpec((_BLK, D), lambda i: (i, 0)),
            pl.BlockSpec((_BLK, 1), lambda i: (i, 0)),
        ],
        out_shape=[
            jax.ShapeDtypeStruct((N, D), jnp.float32),
            jax.ShapeDtypeStruct((N, 1), jnp.float32),
        ],
    )(h, d0, d1)


def _head_body(a0_ref, a1_ref, h_ref, dinv_ref, b1_ref, wm1_ref, bm1_ref,
               wm2_ref, bm2_ref, wa1_ref, wa2_ref, ba_ref,
               f_ref, p_ref, q_ref):
    dinv = dinv_ref[...]
    out = dinv * (a0_ref[...] + a1_ref[...]) + (dinv * dinv) * h_ref[...] + b1_ref[...]
    t = jnp.maximum(
        jnp.dot(out, wm1_ref[...], preferred_element_type=jnp.float32) + bm1_ref[...],
        0.0,
    )
    f_ref[...] = jax.nn.sigmoid(
        jnp.dot(t, wm2_ref[...], preferred_element_type=jnp.float32) + bm2_ref[...]
    )
    p_ref[...] = jnp.dot(out, wa1_ref[...], preferred_element_type=jnp.float32) + ba_ref[0, 0]
    q_ref[...] = jnp.dot(out, wa2_ref[...], preferred_element_type=jnp.float32)


def _head(a0, a1, h, dinv, b1, wm1, bm1, wm2, bm2, wa1, wa2, ba):
    full = lambda shape: pl.BlockSpec(shape, lambda i: tuple(0 for _ in shape))
    return pl.pallas_call(
        _head_body,
        grid=(N // _BLK,),
        in_specs=[
            pl.BlockSpec((_BLK, D), lambda i: (i, 0)),
            pl.BlockSpec((_BLK, D), lambda i: (i, 0)),
            pl.BlockSpec((_BLK, D), lambda i: (i, 0)),
            pl.BlockSpec((_BLK, 1), lambda i: (i, 0)),
            full((1, D)),
            full((D, D)),
            full((1, D)),
            full((D, D)),
            full((1, D)),
            full((D, 1)),
            full((D, 1)),
            full((1, 1)),
        ],
        out_specs=[
            pl.BlockSpec((_BLK, D), lambda i: (i, 0)),
            pl.BlockSpec((_BLK, 1), lambda i: (i, 0)),
            pl.BlockSpec((_BLK, 1), lambda i: (i, 0)),
        ],
        out_shape=[
            jax.ShapeDtypeStruct((N, D), jnp.float32),
            jax.ShapeDtypeStruct((N, 1), jnp.float32),
            jax.ShapeDtypeStruct((N, 1), jnp.float32),
        ],
    )(a0, a1, h, dinv, b1, wm1, bm1, wm2, bm2, wa1, wa2, ba)


def kernel(x, edge_index, k_hop_edge_index, neg_adj, W1, b1, Wm1, bm1, Wm2, bm2, Wa, ba):
    src = edge_index[0]
    dst = edge_index[1]
    epad = E_PAD - E
    pad_src = jnp.arange(epad, dtype=jnp.int32) % N
    pad_dst = N + jnp.arange(epad, dtype=jnp.int32) % (NP - N)
    src2 = jnp.concatenate([src, pad_src]).reshape(-1, 128)
    dst2 = jnp.concatenate([dst, pad_dst]).reshape(-1, 128)
    spad = S_PAD - S
    pad_ab = jnp.arange(spad, dtype=jnp.int32) % N
    a2 = jnp.concatenate(
        [k_hop_edge_index[0], neg_adj[:, 0], pad_ab]
    ).reshape(-1, 128)
    b2 = jnp.concatenate(
        [k_hop_edge_index[1], neg_adj[:, 1], pad_ab]
    ).reshape(-1, 128)

    h = _mm(x, W1)
    deg_parts = _deg(dst2)
    d0 = deg_parts[0].reshape(NP, 1)
    d1 = deg_parts[1].reshape(NP, 1)
    g, dinv = _gscale(h, d0[:N], d1[:N])
    acc = _msg(g, src2, dst2)
    f, p, q = _head(
        acc[0], acc[1], h, dinv,
        b1.reshape(1, D), Wm1, bm1.reshape(1, D), Wm2, bm2.reshape(1, D),
        Wa[:D], Wa[D:], ba.reshape(1, 1),
    )
    sraw = _score(p.reshape(-1), q.reshape(-1), a2, b2)
    adj_out = _sig(sraw).reshape(-1)[:S]
    return f, adj_out
